# baseline (speedup 1.0000x reference)
import jax
import jax.numpy as jnp
from jax import lax
from jax.experimental import pallas as pl
from jax.experimental.pallas import tpu as pltpu
from jax.experimental.pallas import tpu_sc as plsc

_N_HEAD = 4
_CONCEPTS = 4096
_IN_DIM = 256
_DK = 64
_NQ = 2048

_NC = 2
_NS = 16
_LANES = 16
_NW = _NC * _NS
_RPW = _CONCEPTS // _NW

_BR = 256
_NBUF = 2


_DN = (((1,), (1,)), ((), ()))


def _projk_body(key_ref, wk_ref, kt_ref):
    kt_ref[...] = lax.dot_general(
        wk_ref[...], key_ref[...], _DN, preferred_element_type=jnp.float32)


def _projk(key, Wk):
    return pl.pallas_call(
        _projk_body,
        out_shape=jax.ShapeDtypeStruct((_N_HEAD * _DK, _CONCEPTS), jnp.float32),
    )(key, Wk)


def _sc_body(qt_hbm, q2_hbm, qg_hbm, valid_hbm,
             qt_v, inv_v, idx_v, valid_v, rows_v, sem):
    wid = lax.axis_index("s") * _NC + lax.axis_index("c")
    base = wid * _RPW

    pltpu.sync_copy(qt_hbm, qt_v)

    lanes = lax.iota(jnp.int32, _LANES)
    eqmasks = [lanes == l for l in range(_LANES)]
    neg1 = jnp.full((_LANES,), -1, jnp.int32)
    for g in range(_RPW // _LANES):
        inv_v[pl.ds(g * _LANES, _LANES)] = neg1

    def scan_step(g, carry):
        v = qt_v[pl.ds(g * _LANES, _LANES)]
        inr = (v >= base) & (v < base + _RPW)
        lv = jnp.where(inr, v - base, 0)
        vals = g * _LANES + lanes
        nhit = plsc.all_reduce_population_count(inr)

        @pl.when(nhit[0] > 0)
        def _():
            for l in range(_LANES):
                plsc.store_scatter(inv_v, [lv], vals, mask=inr & eqmasks[l])
        return carry

    lax.fori_loop(0, _NQ // _LANES, scan_step, 0, unroll=4)

    for g in range(_RPW // _LANES):
        sl = pl.ds(g * _LANES, _LANES)
        iv = inv_v[sl]
        ok = iv >= 0
        rloc = g * _LANES + lanes
        idx_v[sl] = jnp.where(ok, iv, (base + rloc) & (_NQ - 1))
        valid_v[sl] = jnp.where(ok, 1.0, 0.0)

    pltpu.async_copy(q2_hbm.at[idx_v], rows_v, sem).wait()

    pltpu.sync_copy(valid_v, valid_hbm.at[pl.ds(base, _RPW)])
    pltpu.sync_copy(rows_v, qg_hbm.at[pl.ds(base, _RPW), :])


def _sc_gather(qt, q2):
    mesh = plsc.VectorSubcoreMesh(
        core_axis_name="c", subcore_axis_name="s",
        num_cores=_NC, num_subcores=_NS)
    return pl.kernel(
        _sc_body,
        out_type=(
            jax.ShapeDtypeStruct((_CONCEPTS, _IN_DIM), jnp.float32),
            jax.ShapeDtypeStruct((_CONCEPTS,), jnp.float32),
        ),
        mesh=mesh,
        compiler_params=pltpu.CompilerParams(needs_layout_passes=False),
        scratch_types=[
            pltpu.VMEM((_NQ,), jnp.int32),
            pltpu.VMEM((_RPW,), jnp.int32),
            pltpu.VMEM((_RPW,), jnp.int32),
            pltpu.VMEM((_RPW,), jnp.float32),
            pltpu.VMEM((_RPW, _IN_DIM), jnp.float32),
            pltpu.SemaphoreType.DMA,
        ],
    )(qt, q2)


def _attn_body(qg_ref, wq_ref, kt_ref, valid_ref, out_ref, scr, sems):
    inv_sqrt_dk = 1.0 / (_DK ** 0.5)
    b = pl.program_id(0)
    nb = pl.num_programs(0)
    buf = lax.rem(b, _NBUF)
    v = valid_ref[...].reshape(_BR, 1)
    qg = lax.dot_general(qg_ref[...], wq_ref[...], _DN,
                         preferred_element_type=jnp.float32)

    def _copy(bb, hh, step):
        return pltpu.make_async_copy(
            scr.at[bb, hh],
            out_ref.at[hh, pl.ds(step * _BR, _BR), :],
            sems.at[bb, hh])

    for h in range(_N_HEAD):
        q = qg[:, h * _DK:(h + 1) * _DK]
        k = kt_ref[h]
        e = jnp.exp(
            jnp.dot(q, k, preferred_element_type=jnp.float32) * inv_sqrt_dk)
        den = jnp.sum(e, axis=1, keepdims=True)

        @pl.when(b >= _NBUF)
        def _():
            _copy(buf, h, b - _NBUF).wait()
        scr[buf, h] = e * (v / den)
        _copy(buf, h, b).start()

    @pl.when(b == nb - 1)
    def _():
        for off in range(_NBUF - 1, -1, -1):
            s = b - off
            bb = lax.rem(s, _NBUF)
            for h in range(_N_HEAD):
                _copy(bb, h, s).wait()


def _attn(qg, Wq, kt3, valid2):
    return pl.pallas_call(
        _attn_body,
        grid=(_CONCEPTS // _BR,),
        in_specs=[
            pl.BlockSpec((_BR, _IN_DIM), lambda b: (b, 0)),
            pl.BlockSpec((_IN_DIM, _IN_DIM), lambda b: (0, 0)),
            pl.BlockSpec((_N_HEAD, _DK, _CONCEPTS), lambda b: (0, 0, 0)),
            pl.BlockSpec((_BR,), lambda b: (b,)),
        ],
        out_specs=pl.BlockSpec(memory_space=pl.ANY),
        out_shape=jax.ShapeDtypeStruct(
            (_N_HEAD, _CONCEPTS, _CONCEPTS), jnp.float32),
        scratch_shapes=[
            pltpu.VMEM((_NBUF, _N_HEAD, _BR, _CONCEPTS), jnp.float32),
            pltpu.SemaphoreType.DMA((_NBUF, _N_HEAD)),
        ],
    )(qg, Wq, kt3, valid2)


@jax.jit
def kernel(qt, query, key, Wq, Wk):
    qt = qt.astype(jnp.int32)
    qg, valid = _sc_gather(qt, query)
    kt = _projk(key, Wk)
    kt3 = kt.reshape(_N_HEAD, _DK, _CONCEPTS)
    return _attn(qg, Wq, kt3, valid)


# speedup vs baseline: 1.0217x; 1.0217x over previous
import jax
import jax.numpy as jnp
from jax import lax
from jax.experimental import pallas as pl
from jax.experimental.pallas import tpu as pltpu
from jax.experimental.pallas import tpu_sc as plsc

_N_HEAD = 4
_CONCEPTS = 4096
_IN_DIM = 256
_DK = 64
_NQ = 2048

_NC = 2
_NS = 16
_LANES = 16
_NW = _NC * _NS
_RPW = _CONCEPTS // _NW

_BR = 256
_NBUF = 2


_DN = (((1,), (1,)), ((), ()))


def _projk_body(key_ref, wk_ref, kt_ref):
    kt_ref[...] = lax.dot_general(
        wk_ref[...], key_ref[...], _DN, preferred_element_type=jnp.float32)


def _projk(key, Wk):
    return pl.pallas_call(
        _projk_body,
        out_shape=jax.ShapeDtypeStruct((_N_HEAD * _DK, _CONCEPTS), jnp.float32),
    )(key, Wk)


def _sc_body(qt_hbm, q2_hbm, qg_hbm, valid_hbm,
             qt_v, inv_v, idx_v, valid_v, rows_v, sem):
    wid = lax.axis_index("s") * _NC + lax.axis_index("c")
    base = wid * _RPW

    pltpu.sync_copy(qt_hbm, qt_v)

    lanes = lax.iota(jnp.int32, _LANES)
    eqmasks = [lanes == l for l in range(_LANES)]
    neg1 = jnp.full((_LANES,), -1, jnp.int32)
    for g in range(_RPW // _LANES):
        inv_v[pl.ds(g * _LANES, _LANES)] = neg1

    def scan_step(g, carry):
        v = qt_v[pl.ds(g * _LANES, _LANES)]
        inr = (v >= base) & (v < base + _RPW)
        lv = jnp.where(inr, v - base, 0)
        vals = g * _LANES + lanes
        plsc.store_scatter(inv_v, [lv], vals, mask=inr)
        return carry

    lax.fori_loop(0, _NQ // _LANES, scan_step, 0, unroll=4)

    for g in range(_RPW // _LANES):
        sl = pl.ds(g * _LANES, _LANES)
        iv = inv_v[sl]
        ok = iv >= 0
        rloc = g * _LANES + lanes
        idx_v[sl] = jnp.where(ok, iv, (base + rloc) & (_NQ - 1))
        valid_v[sl] = jnp.where(ok, 1.0, 0.0)

    pltpu.async_copy(q2_hbm.at[idx_v], rows_v, sem).wait()

    pltpu.sync_copy(valid_v, valid_hbm.at[pl.ds(base, _RPW)])
    pltpu.sync_copy(rows_v, qg_hbm.at[pl.ds(base, _RPW), :])


def _sc_gather(qt, q2):
    mesh = plsc.VectorSubcoreMesh(
        core_axis_name="c", subcore_axis_name="s",
        num_cores=_NC, num_subcores=_NS)
    return pl.kernel(
        _sc_body,
        out_type=(
            jax.ShapeDtypeStruct((_CONCEPTS, _IN_DIM), jnp.float32),
            jax.ShapeDtypeStruct((_CONCEPTS,), jnp.float32),
        ),
        mesh=mesh,
        compiler_params=pltpu.CompilerParams(needs_layout_passes=False),
        scratch_types=[
            pltpu.VMEM((_NQ,), jnp.int32),
            pltpu.VMEM((_RPW,), jnp.int32),
            pltpu.VMEM((_RPW,), jnp.int32),
            pltpu.VMEM((_RPW,), jnp.float32),
            pltpu.VMEM((_RPW, _IN_DIM), jnp.float32),
            pltpu.SemaphoreType.DMA,
        ],
    )(qt, q2)


def _attn_body(qg_ref, wq_ref, kt_ref, valid_ref, out_ref, scr, sems):
    inv_sqrt_dk = 1.0 / (_DK ** 0.5)
    b = pl.program_id(0)
    nb = pl.num_programs(0)
    buf = lax.rem(b, _NBUF)
    v = valid_ref[...].reshape(_BR, 1)
    qg = lax.dot_general(qg_ref[...], wq_ref[...], _DN,
                         preferred_element_type=jnp.float32)

    def _copy(bb, hh, step):
        return pltpu.make_async_copy(
            scr.at[bb, hh],
            out_ref.at[hh, pl.ds(step * _BR, _BR), :],
            sems.at[bb, hh])

    for h in range(_N_HEAD):
        q = qg[:, h * _DK:(h + 1) * _DK]
        k = kt_ref[h]
        e = jnp.exp(
            jnp.dot(q, k, preferred_element_type=jnp.float32) * inv_sqrt_dk)
        den = jnp.sum(e, axis=1, keepdims=True)

        @pl.when(b >= _NBUF)
        def _():
            _copy(buf, h, b - _NBUF).wait()
        scr[buf, h] = e * (v / den)
        _copy(buf, h, b).start()

    @pl.when(b == nb - 1)
    def _():
        for off in range(_NBUF - 1, -1, -1):
            s = b - off
            bb = lax.rem(s, _NBUF)
            for h in range(_N_HEAD):
                _copy(bb, h, s).wait()


def _attn(qg, Wq, kt3, valid2):
    return pl.pallas_call(
        _attn_body,
        grid=(_CONCEPTS // _BR,),
        in_specs=[
            pl.BlockSpec((_BR, _IN_DIM), lambda b: (b, 0)),
            pl.BlockSpec((_IN_DIM, _IN_DIM), lambda b: (0, 0)),
            pl.BlockSpec((_N_HEAD, _DK, _CONCEPTS), lambda b: (0, 0, 0)),
            pl.BlockSpec((_BR,), lambda b: (b,)),
        ],
        out_specs=pl.BlockSpec(memory_space=pl.ANY),
        out_shape=jax.ShapeDtypeStruct(
            (_N_HEAD, _CONCEPTS, _CONCEPTS), jnp.float32),
        scratch_shapes=[
            pltpu.VMEM((_NBUF, _N_HEAD, _BR, _CONCEPTS), jnp.float32),
            pltpu.SemaphoreType.DMA((_NBUF, _N_HEAD)),
        ],
    )(qg, Wq, kt3, valid2)


@jax.jit
def kernel(qt, query, key, Wq, Wk):
    qt = qt.astype(jnp.int32)
    qg, valid = _sc_gather(qt, query)
    kt = _projk(key, Wk)
    kt3 = kt.reshape(_N_HEAD, _DK, _CONCEPTS)
    return _attn(qg, Wq, kt3, valid)
